# baseline (speedup 1.0000x reference)
import jax
import jax.numpy as jnp
from jax import lax
from jax.experimental import pallas as pl
from jax.experimental.pallas import tpu as pltpu
from jax.experimental.pallas import tpu_sc as plsc

_NUM_WORKERS = 32


def _sc_copy_body(src_hbm, out_hbm, sem):
    seq_len = out_hbm.shape[0]
    rows_per_worker = seq_len // _NUM_WORKERS
    wid = lax.axis_index("s") * 2 + lax.axis_index("c")
    base = wid * rows_per_worker
    pltpu.async_copy(
        src_hbm.at[pl.ds(base, rows_per_worker), :],
        out_hbm.at[pl.ds(base, rows_per_worker), :],
        sem,
    ).wait()


def kernel(positions, pos_embedding):
    seq_len = positions.shape[1]
    d_model = pos_embedding.shape[2]
    table = pos_embedding.reshape(pos_embedding.shape[1], d_model)
    mesh = plsc.VectorSubcoreMesh(core_axis_name="c", subcore_axis_name="s")
    copy = pl.kernel(
        _sc_copy_body,
        out_type=jax.ShapeDtypeStruct((seq_len, d_model), pos_embedding.dtype),
        mesh=mesh,
        scratch_types=[pltpu.SemaphoreType.DMA],
    )
    out = copy(table)
    return out.reshape(1, seq_len, d_model)


# speedup vs baseline: 16.6221x; 16.6221x over previous
import jax
import jax.numpy as jnp
from jax import lax
from jax.experimental import pallas as pl
from jax.experimental.pallas import tpu as pltpu
from jax.experimental.pallas import tpu_sc as plsc

_NUM_WORKERS = 32
_CHUNK_ROWS = 32


def _sc_copy_body(src_hbm, out_hbm, buf0, buf1, isem0, isem1, osem0, osem1):
    seq_len = out_hbm.shape[0]
    rows_per_worker = seq_len // _NUM_WORKERS
    nchunks = rows_per_worker // _CHUNK_ROWS
    wid = lax.axis_index("s") * 2 + lax.axis_index("c")
    base = wid * rows_per_worker

    bufs = (buf0, buf1)
    isems = (isem0, isem1)
    osems = (osem0, osem1)

    in_copies = [None] * nchunks
    out_copies = [None] * nchunks
    for c in range(min(2, nchunks)):
        in_copies[c] = pltpu.async_copy(
            src_hbm.at[pl.ds(base + c * _CHUNK_ROWS, _CHUNK_ROWS), :],
            bufs[c % 2], isems[c % 2])
    for c in range(nchunks):
        b = c % 2
        if c >= 2:
            out_copies[c - 2].wait()
            in_copies[c] = pltpu.async_copy(
                src_hbm.at[pl.ds(base + c * _CHUNK_ROWS, _CHUNK_ROWS), :],
                bufs[b], isems[b])
        in_copies[c].wait()
        out_copies[c] = pltpu.async_copy(
            bufs[b],
            out_hbm.at[pl.ds(base + c * _CHUNK_ROWS, _CHUNK_ROWS), :],
            osems[b])
    for c in range(max(0, nchunks - 2), nchunks):
        out_copies[c].wait()


def kernel(positions, pos_embedding):
    seq_len = positions.shape[1]
    d_model = pos_embedding.shape[2]
    table = pos_embedding.reshape(pos_embedding.shape[1], d_model)
    mesh = plsc.VectorSubcoreMesh(core_axis_name="c", subcore_axis_name="s")
    copy = pl.kernel(
        _sc_copy_body,
        out_type=jax.ShapeDtypeStruct((seq_len, d_model), pos_embedding.dtype),
        mesh=mesh,
        scratch_types=[
            pltpu.VMEM((_CHUNK_ROWS, d_model), jnp.float32),
            pltpu.VMEM((_CHUNK_ROWS, d_model), jnp.float32),
            pltpu.SemaphoreType.DMA,
            pltpu.SemaphoreType.DMA,
            pltpu.SemaphoreType.DMA,
            pltpu.SemaphoreType.DMA,
        ],
    )
    out = copy(table)
    return out.reshape(1, seq_len, d_model)


# speedup vs baseline: 42.3633x; 2.5486x over previous
import jax
import jax.numpy as jnp
from jax.experimental import pallas as pl
from jax.experimental.pallas import tpu as pltpu

_BLOCK_ROWS = 1024


def _copy_kernel(src_ref, dst_ref):
    dst_ref[...] = src_ref[...]


def kernel(positions, pos_embedding):
    seq_len = positions.shape[1]
    d_model = pos_embedding.shape[2]
    grid = (seq_len // _BLOCK_ROWS,)
    return pl.pallas_call(
        _copy_kernel,
        grid=grid,
        out_shape=jax.ShapeDtypeStruct((1, seq_len, d_model), pos_embedding.dtype),
        in_specs=[
            pl.BlockSpec((1, _BLOCK_ROWS, d_model), lambda i: (0, i, 0)),
        ],
        out_specs=pl.BlockSpec((1, _BLOCK_ROWS, d_model), lambda i: (0, i, 0)),
    )(pos_embedding)


# speedup vs baseline: 46.9580x; 1.1085x over previous
import jax
import jax.numpy as jnp
from jax.experimental import pallas as pl
from jax.experimental.pallas import tpu as pltpu

_BLOCK_ROWS = 2048


def _copy_kernel(src_ref, dst_ref):
    dst_ref[...] = src_ref[...]


def kernel(positions, pos_embedding):
    seq_len = positions.shape[1]
    d_model = pos_embedding.shape[2]
    grid = (seq_len // _BLOCK_ROWS,)
    return pl.pallas_call(
        _copy_kernel,
        grid=grid,
        out_shape=jax.ShapeDtypeStruct((1, seq_len, d_model), pos_embedding.dtype),
        in_specs=[
            pl.BlockSpec((1, _BLOCK_ROWS, d_model), lambda i: (0, i, 0)),
        ],
        out_specs=pl.BlockSpec((1, _BLOCK_ROWS, d_model), lambda i: (0, i, 0)),
    )(pos_embedding)
